# Initial kernel scaffold; baseline (speedup 1.0000x reference)
import jax
import jax.numpy as jnp
from jax.experimental import pallas as pl


def kernel(X, A, Wn1, bn1, W1, b1, Wn2, bn2, W2, b2, W3, b3):
    raise NotImplementedError("write your pallas kernel here")


# speedup vs baseline: 3.4776x; 3.4776x over previous
import functools

import jax
import jax.numpy as jnp
from jax.experimental import pallas as pl

_N = 4096
_F = 256
_C = 40
_NB1 = 25
_NB2 = 10
_BM = 256
_CK = 256


def _lrelu(x):
    return jnp.where(x >= 0, x, 0.01 * x)


def _sel_mean(a, x_ref, nb):
    maskf = (a != 0).astype(jnp.float32)
    r = jax.lax.broadcasted_iota(jnp.int32, (_CK, _CK), 0)
    c = jax.lax.broadcasted_iota(jnp.int32, (_CK, _CK), 1)
    tri = (r <= c).astype(jnp.bfloat16)
    carry = jnp.zeros((_BM, 1), jnp.float32)
    acc = jnp.zeros((_BM, _F), jnp.float32)
    for ci in range(_N // _CK):
        mc = maskf[:, ci * _CK:(ci + 1) * _CK]
        csum = jax.lax.dot_general(
            mc.astype(jnp.bfloat16), tri, (((1,), (0,)), ((), ())),
            preferred_element_type=jnp.float32) + carry
        sel = jnp.where((mc != 0) & (csum <= nb), 1.0, 0.0)
        xc = x_ref[pl.ds(ci * _CK, _CK), :]
        acc = acc + jax.lax.dot_general(
            sel, xc, (((1,), (0,)), ((), ())),
            preferred_element_type=jnp.float32)
        carry = carry + jnp.sum(mc, axis=1, keepdims=True)
    cnt = jnp.minimum(carry, float(nb))
    mean = acc / jnp.maximum(cnt, 1.0)
    return mean, carry


def _layer1_body(a_ref, x_ref, xb_ref, wnT_ref, bn_ref, wT_ref, b_ref, o_ref):
    mean, total = _sel_mean(a_ref[...], x_ref, _NB1)
    xj = _lrelu(jax.lax.dot_general(
        mean, wnT_ref[...], (((1,), (0,)), ((), ())),
        preferred_element_type=jnp.float32) + bn_ref[...])
    xi = _lrelu(jax.lax.dot_general(
        xb_ref[...], wT_ref[...], (((1,), (0,)), ((), ())),
        preferred_element_type=jnp.float32) + b_ref[...])
    o_ref[...] = xi + jnp.where(total > 0, xj, 0.0)


def _layer2_body(a_ref, h_ref, hb_ref, wnT_ref, bn_ref, wT_ref, b_ref,
                 w3T_ref, b3_ref, o_ref):
    mean, total = _sel_mean(a_ref[...], h_ref, _NB2)
    xj = _lrelu(jax.lax.dot_general(
        mean, wnT_ref[...], (((1,), (0,)), ((), ())),
        preferred_element_type=jnp.float32) + bn_ref[...])
    xi = _lrelu(jax.lax.dot_general(
        hb_ref[...], wT_ref[...], (((1,), (0,)), ((), ())),
        preferred_element_type=jnp.float32) + b_ref[...])
    h2 = xi + jnp.where(total > 0, xj, 0.0)
    logits = jax.lax.dot_general(
        h2, w3T_ref[...], (((1,), (0,)), ((), ())),
        preferred_element_type=jnp.float32) + b3_ref[...]
    m = jnp.max(logits, axis=1, keepdims=True)
    shifted = logits - m
    lse = jnp.log(jnp.sum(jnp.exp(shifted), axis=1, keepdims=True))
    o_ref[...] = shifted - lse


def _full(shape):
    return pl.BlockSpec(shape, lambda i: (0, 0))


def kernel(X, A, Wn1, bn1, W1, b1, Wn2, bn2, W2, b2, W3, b3):
    grid = (_N // _BM,)
    row_block = lambda i: (i, 0)

    h = pl.pallas_call(
        _layer1_body,
        grid=grid,
        in_specs=[
            pl.BlockSpec((_BM, _N), row_block),
            _full((_N, _F)),
            pl.BlockSpec((_BM, _F), row_block),
            _full((_F, _F)),
            _full((1, _F)),
            _full((_F, _F)),
            _full((1, _F)),
        ],
        out_specs=pl.BlockSpec((_BM, _F), row_block),
        out_shape=jax.ShapeDtypeStruct((_N, _F), jnp.float32),
    )(A, X, X, Wn1.T, bn1[None, :], W1.T, b1[None, :])

    out = pl.pallas_call(
        _layer2_body,
        grid=grid,
        in_specs=[
            pl.BlockSpec((_BM, _N), row_block),
            _full((_N, _F)),
            pl.BlockSpec((_BM, _F), row_block),
            _full((_F, _F)),
            _full((1, _F)),
            _full((_F, _F)),
            _full((1, _F)),
            _full((_F, _C)),
            _full((1, _C)),
        ],
        out_specs=pl.BlockSpec((_BM, _C), row_block),
        out_shape=jax.ShapeDtypeStruct((_N, _C), jnp.float32),
    )(A, h, h, Wn2.T, bn2[None, :], W2.T, b2[None, :], W3.T, b3[None, :])
    return out
